# baseline (speedup 1.0000x reference)
import functools

import jax
import jax.numpy as jnp
from jax import lax
from jax.experimental import pallas as pl
from jax.experimental.pallas import tpu as pltpu
from jax.experimental.pallas import tpu_sc as plsc

DIM = 64
BATCH = 16384
PAIR = 2 * DIM

NC = 2
NS = 16
NW = NC * NS
B_PER_W = BATCH // NW
CHUNK = 128
N_CHUNK = B_PER_W // CHUNK


@functools.cache
def _make_sc_gather():
    mesh = plsc.VectorSubcoreMesh(core_axis_name="c", subcore_axis_name="s")

    @functools.partial(
        pl.kernel,
        mesh=mesh,
        out_type=jax.ShapeDtypeStruct((BATCH, PAIR), jnp.float32),
        scratch_types=[
            pltpu.VMEM((N_CHUNK, CHUNK), jnp.int32),
            pltpu.VMEM((B_PER_W, PAIR), jnp.float32),
            pltpu.SemaphoreType.DMA,
        ],
        compiler_params=pltpu.CompilerParams(needs_layout_passes=False),
    )
    def _sc_gather(idx_hbm, table_hbm, out_hbm, idx_v, rows_v, sem):
        wid = lax.axis_index("s") * NC + lax.axis_index("c")
        base = wid * B_PER_W
        pltpu.sync_copy(idx_hbm.at[pl.ds(wid * N_CHUNK, N_CHUNK)], idx_v)
        lane = lax.iota(jnp.int32, 16)

        def body(g, _):
            j = g // 8
            k = (g % 8) * 16
            v = idx_v[j, pl.ds(k, 16)]
            for l in range(16):
                s = jnp.sum(jnp.where(lane == l, v, 0))
                pltpu.async_copy(
                    table_hbm.at[s // 2],
                    rows_v.at[g * 16 + l],
                    sem,
                )
            return 0

        lax.fori_loop(0, B_PER_W // 16, body, 0)
        pltpu.make_async_copy(
            out_hbm.at[pl.ds(0, B_PER_W)], rows_v, sem
        ).wait()
        pltpu.sync_copy(rows_v, out_hbm.at[pl.ds(base, B_PER_W)])

    return _sc_gather


_MLP_BLK = 2048


def _mlp_body(hp_ref, xr_ref, w1_ref, b1_ref, w2_ref, b2_ref, o_ref):
    odd = (xr_ref[...] & 1) == 1
    h = jnp.where(odd, hp_ref[:, DIM:], hp_ref[:, :DIM])
    z = jax.lax.dot_general(
        h, w1_ref[...], (((1,), (1,)), ((), ())),
        preferred_element_type=jnp.float32,
    ) + b1_ref[...]
    z = z * jax.nn.sigmoid(z)
    o_ref[...] = jax.lax.dot_general(
        z, w2_ref[...], (((1,), (1,)), ((), ())),
        preferred_element_type=jnp.float32,
    ) + b2_ref[...]


def _mlp(hp, xr, w1, b1, w2, b2):
    grid = (BATCH // _MLP_BLK,)
    return pl.pallas_call(
        _mlp_body,
        grid=grid,
        in_specs=[
            pl.BlockSpec((_MLP_BLK, PAIR), lambda i: (i, 0)),
            pl.BlockSpec((_MLP_BLK, 1), lambda i: (i, 0)),
            pl.BlockSpec((DIM, DIM), lambda i: (0, 0)),
            pl.BlockSpec((1, DIM), lambda i: (0, 0)),
            pl.BlockSpec((DIM, DIM), lambda i: (0, 0)),
            pl.BlockSpec((1, DIM), lambda i: (0, 0)),
        ],
        out_specs=pl.BlockSpec((_MLP_BLK, DIM), lambda i: (i, 0)),
        out_shape=jax.ShapeDtypeStruct((BATCH, DIM), jnp.float32),
    )(hp, xr, w1, b1.reshape(1, DIM), w2, b2.reshape(1, DIM))


def kernel(x, emb, W1, b1, W2, b2):
    idx = x.astype(jnp.int32)
    table = emb.reshape(500000, PAIR)
    hp = _make_sc_gather()(idx.reshape(NW * N_CHUNK, CHUNK), table)
    return _mlp(hp, idx.reshape(BATCH, 1), W1, b1, W2, b2)


# speedup vs baseline: 1.6945x; 1.6945x over previous
import functools

import jax
import jax.numpy as jnp
from jax import lax
from jax.experimental import pallas as pl
from jax.experimental.pallas import tpu as pltpu
from jax.experimental.pallas import tpu_sc as plsc

DIM = 64
BATCH = 16384
NUM_CLASSES = 1000000
PAIR = 2 * DIM
N_C = 500032

NC = 2
NS = 16
NW = NC * NS
B_PER_W = BATCH // NW
CHUNK = 128
N_CHUNK = B_PER_W // CHUNK

_T_LANES = 4096
_T_GRID = 245


def _transpose_body(xt_ref, o_ref):
    eye = jnp.eye(DIM, dtype=jnp.float32)
    for t in range(_T_LANES // 256):
        y = jax.lax.dot_general(
            xt_ref[:, pl.ds(t * 256, 256)], eye, (((0,), (0,)), ((), ())),
            preferred_element_type=jnp.float32,
        )
        o_ref[pl.ds(t * 128, 128), :DIM] = y[:128]
        o_ref[pl.ds(t * 128, 128), DIM:] = y[128:]


def _compact_transpose(emb_t):
    return pl.pallas_call(
        _transpose_body,
        grid=(_T_GRID,),
        in_specs=[pl.BlockSpec((DIM, _T_LANES), lambda m: (0, m))],
        out_specs=pl.BlockSpec((_T_LANES // 2, PAIR), lambda m: (m, 0)),
        out_shape=jax.ShapeDtypeStruct((N_C, PAIR), jnp.float32),
    )(emb_t)


@functools.cache
def _make_sc_gather():
    mesh = plsc.VectorSubcoreMesh(core_axis_name="c", subcore_axis_name="s")

    @functools.partial(
        pl.kernel,
        mesh=mesh,
        out_type=jax.ShapeDtypeStruct((BATCH, PAIR), jnp.float32),
        scratch_types=[
            pltpu.VMEM((N_CHUNK, CHUNK), jnp.int32),
            pltpu.VMEM((B_PER_W, PAIR), jnp.float32),
            pltpu.SemaphoreType.DMA,
        ],
        compiler_params=pltpu.CompilerParams(needs_layout_passes=False),
    )
    def _sc_gather(idx_hbm, table_hbm, out_hbm, idx_v, rows_v, sem):
        wid = lax.axis_index("s") * NC + lax.axis_index("c")
        base = wid * B_PER_W
        pltpu.sync_copy(idx_hbm.at[pl.ds(wid * N_CHUNK, N_CHUNK)], idx_v)
        lane = lax.iota(jnp.int32, 16)

        def body(g, _):
            j = g // 8
            k = (g % 8) * 16
            v = idx_v[j, pl.ds(k, 16)]
            for l in range(16):
                s = jnp.sum(jnp.where(lane == l, v, 0))
                q = ((s >> 8) << 7) | (s & 127)
                pltpu.async_copy(
                    table_hbm.at[q],
                    rows_v.at[g * 16 + l],
                    sem,
                )
            return 0

        lax.fori_loop(0, B_PER_W // 16, body, 0)
        pltpu.make_async_copy(
            out_hbm.at[pl.ds(0, B_PER_W)], rows_v, sem
        ).wait()
        pltpu.sync_copy(rows_v, out_hbm.at[pl.ds(base, B_PER_W)])

    return _sc_gather


_MLP_BLK = 2048


def _mlp_body(hp_ref, xr_ref, w1_ref, b1_ref, w2_ref, b2_ref, o_ref):
    odd = ((xr_ref[...] >> 7) & 1) == 1
    h = jnp.where(odd, hp_ref[:, DIM:], hp_ref[:, :DIM])
    z = jax.lax.dot_general(
        h, w1_ref[...], (((1,), (1,)), ((), ())),
        preferred_element_type=jnp.float32,
    ) + b1_ref[...]
    z = z * jax.nn.sigmoid(z)
    o_ref[...] = jax.lax.dot_general(
        z, w2_ref[...], (((1,), (1,)), ((), ())),
        preferred_element_type=jnp.float32,
    ) + b2_ref[...]


def _mlp(hp, xr, w1, b1, w2, b2):
    grid = (BATCH // _MLP_BLK,)
    return pl.pallas_call(
        _mlp_body,
        grid=grid,
        in_specs=[
            pl.BlockSpec((_MLP_BLK, PAIR), lambda i: (i, 0)),
            pl.BlockSpec((_MLP_BLK, 1), lambda i: (i, 0)),
            pl.BlockSpec((DIM, DIM), lambda i: (0, 0)),
            pl.BlockSpec((1, DIM), lambda i: (0, 0)),
            pl.BlockSpec((DIM, DIM), lambda i: (0, 0)),
            pl.BlockSpec((1, DIM), lambda i: (0, 0)),
        ],
        out_specs=pl.BlockSpec((_MLP_BLK, DIM), lambda i: (i, 0)),
        out_shape=jax.ShapeDtypeStruct((BATCH, DIM), jnp.float32),
    )(hp, xr, w1, b1.reshape(1, DIM), w2, b2.reshape(1, DIM))


def kernel(x, emb, W1, b1, W2, b2):
    idx = x.astype(jnp.int32)
    table = _compact_transpose(emb.T)
    hp = _make_sc_gather()(idx.reshape(NW * N_CHUNK, CHUNK), table)
    return _mlp(hp, idx.reshape(BATCH, 1), W1, b1, W2, b2)


# speedup vs baseline: 2.1101x; 1.2452x over previous
import functools

import jax
import jax.numpy as jnp
from jax import lax
from jax.experimental import pallas as pl
from jax.experimental.pallas import tpu as pltpu
from jax.experimental.pallas import tpu_sc as plsc

DIM = 64
BATCH = 16384
NUM_CLASSES = 1000000
PAIR = 2 * DIM
N_C = 500032

NC = 2
NS = 16
NW = NC * NS
B_PER_W = BATCH // NW
CHUNK = 128
N_CHUNK = B_PER_W // CHUNK

_T_LANES = 8192
_T_GRID = 123


def _transpose_body(xt_ref, o_ref):
    eye = jnp.eye(DIM, dtype=jnp.float32)
    for t in range(_T_LANES // 512):
        y = jax.lax.dot_general(
            xt_ref[:, pl.ds(t * 512, 512)], eye, (((0,), (0,)), ((), ())),
            preferred_element_type=jnp.float32,
        )
        o_ref[pl.ds(t * 256, 128), :] = jnp.concatenate(
            [y[:128], y[128:256]], axis=1
        )
        o_ref[pl.ds(t * 256 + 128, 128), :] = jnp.concatenate(
            [y[256:384], y[384:]], axis=1
        )


def _compact_transpose(emb_t):
    return pl.pallas_call(
        _transpose_body,
        grid=(_T_GRID,),
        in_specs=[pl.BlockSpec((DIM, _T_LANES), lambda m: (0, m))],
        out_specs=pl.BlockSpec((_T_LANES // 2, PAIR), lambda m: (m, 0)),
        compiler_params=pltpu.CompilerParams(
            dimension_semantics=("arbitrary",)
        ),
        out_shape=jax.ShapeDtypeStruct((N_C, PAIR), jnp.float32),
    )(emb_t)


@functools.cache
def _make_sc_gather():
    mesh = plsc.VectorSubcoreMesh(core_axis_name="c", subcore_axis_name="s")

    @functools.partial(
        pl.kernel,
        mesh=mesh,
        out_type=jax.ShapeDtypeStruct((BATCH, PAIR), jnp.float32),
        scratch_types=[
            pltpu.VMEM((N_CHUNK, CHUNK), jnp.int32),
            pltpu.VMEM((B_PER_W, PAIR), jnp.float32),
            pltpu.SemaphoreType.DMA,
        ],
        compiler_params=pltpu.CompilerParams(needs_layout_passes=False),
    )
    def _sc_gather(idx_hbm, table_hbm, out_hbm, idx_v, rows_v, sem):
        wid = lax.axis_index("s") * NC + lax.axis_index("c")
        base = wid * B_PER_W
        pltpu.sync_copy(idx_hbm.at[pl.ds(wid * N_CHUNK, N_CHUNK)], idx_v)
        lane = lax.iota(jnp.int32, 16)

        def body(g, _):
            j = g // 8
            k = (g % 8) * 16
            v = idx_v[j, pl.ds(k, 16)]
            for l in range(16):
                s = jnp.sum(jnp.where(lane == l, v, 0))
                q = ((s >> 8) << 7) | (s & 127)
                pltpu.async_copy(
                    table_hbm.at[q],
                    rows_v.at[g * 16 + l],
                    sem,
                )
            return 0

        lax.fori_loop(0, B_PER_W // 16, body, 0)
        pltpu.make_async_copy(
            out_hbm.at[pl.ds(0, B_PER_W)], rows_v, sem
        ).wait()
        pltpu.sync_copy(rows_v, out_hbm.at[pl.ds(base, B_PER_W)])

    return _sc_gather


_MLP_BLK = 2048


def _mlp_body(hp_ref, xr_ref, w1_ref, b1_ref, w2_ref, b2_ref, o_ref):
    odd = ((xr_ref[...] >> 7) & 1) == 1
    h = jnp.where(odd, hp_ref[:, DIM:], hp_ref[:, :DIM])
    z = jax.lax.dot_general(
        h, w1_ref[...], (((1,), (1,)), ((), ())),
        preferred_element_type=jnp.float32,
    ) + b1_ref[...]
    z = z * jax.nn.sigmoid(z)
    o_ref[...] = jax.lax.dot_general(
        z, w2_ref[...], (((1,), (1,)), ((), ())),
        preferred_element_type=jnp.float32,
    ) + b2_ref[...]


def _mlp(hp, xr, w1, b1, w2, b2):
    grid = (BATCH // _MLP_BLK,)
    return pl.pallas_call(
        _mlp_body,
        grid=grid,
        in_specs=[
            pl.BlockSpec((_MLP_BLK, PAIR), lambda i: (i, 0)),
            pl.BlockSpec((_MLP_BLK, 1), lambda i: (i, 0)),
            pl.BlockSpec((DIM, DIM), lambda i: (0, 0)),
            pl.BlockSpec((1, DIM), lambda i: (0, 0)),
            pl.BlockSpec((DIM, DIM), lambda i: (0, 0)),
            pl.BlockSpec((1, DIM), lambda i: (0, 0)),
        ],
        out_specs=pl.BlockSpec((_MLP_BLK, DIM), lambda i: (i, 0)),
        out_shape=jax.ShapeDtypeStruct((BATCH, DIM), jnp.float32),
    )(hp, xr, w1, b1.reshape(1, DIM), w2, b2.reshape(1, DIM))


def kernel(x, emb, W1, b1, W2, b2):
    idx = x.astype(jnp.int32)
    table = _compact_transpose(emb.T)
    hp = _make_sc_gather()(idx.reshape(NW * N_CHUNK, CHUNK), table)
    return _mlp(hp, idx.reshape(BATCH, 1), W1, b1, W2, b2)


# speedup vs baseline: 2.3701x; 1.1232x over previous
import functools

import jax
import jax.numpy as jnp
from jax import lax
from jax.experimental import pallas as pl
from jax.experimental.pallas import tpu as pltpu
from jax.experimental.pallas import tpu_sc as plsc

DIM = 64
BATCH = 16384
NUM_CLASSES = 1000000
PAIR = 2 * DIM
N_C = 500032

NC = 2
NS = 16
NW = NC * NS
B_PER_W = BATCH // NW
CHUNK = 128
N_CHUNK = B_PER_W // CHUNK

_T_LANES = 16384
_T_GRID = 62


def _transpose_body(xt_ref, o_ref):
    eye = jnp.eye(DIM, dtype=jnp.float32)
    for t in range(_T_LANES // 512):
        y = jax.lax.dot_general(
            xt_ref[:, pl.ds(t * 512, 512)], eye, (((0,), (0,)), ((), ())),
            preferred_element_type=jnp.float32,
        )
        o_ref[pl.ds(t * 256, 128), :] = jnp.concatenate(
            [y[:128], y[128:256]], axis=1
        )
        o_ref[pl.ds(t * 256 + 128, 128), :] = jnp.concatenate(
            [y[256:384], y[384:]], axis=1
        )


def _compact_transpose(emb_t):
    return pl.pallas_call(
        _transpose_body,
        grid=(_T_GRID,),
        in_specs=[pl.BlockSpec((DIM, _T_LANES), lambda m: (0, m))],
        out_specs=pl.BlockSpec((_T_LANES // 2, PAIR), lambda m: (m, 0)),
        compiler_params=pltpu.CompilerParams(
            dimension_semantics=("arbitrary",)
        ),
        out_shape=jax.ShapeDtypeStruct((N_C, PAIR), jnp.float32),
    )(emb_t)


@functools.cache
def _make_sc_gather():
    mesh = plsc.VectorSubcoreMesh(core_axis_name="c", subcore_axis_name="s")

    @functools.partial(
        pl.kernel,
        mesh=mesh,
        out_type=jax.ShapeDtypeStruct((BATCH, PAIR), jnp.float32),
        scratch_types=[
            pltpu.VMEM((N_CHUNK, CHUNK), jnp.int32),
            pltpu.VMEM((B_PER_W, PAIR), jnp.float32),
            pltpu.SemaphoreType.DMA,
        ],
        compiler_params=pltpu.CompilerParams(needs_layout_passes=False),
    )
    def _sc_gather(idx_hbm, table_hbm, out_hbm, idx_v, rows_v, sem):
        wid = lax.axis_index("s") * NC + lax.axis_index("c")
        base = wid * B_PER_W
        pltpu.sync_copy(idx_hbm.at[pl.ds(wid * N_CHUNK, N_CHUNK)], idx_v)
        lane = lax.iota(jnp.int32, 16)

        def body(g, _):
            j = g // 8
            k = (g % 8) * 16
            v = idx_v[j, pl.ds(k, 16)]
            for l in range(16):
                s = jnp.sum(jnp.where(lane == l, v, 0))
                q = ((s >> 8) << 7) | (s & 127)
                pltpu.async_copy(
                    table_hbm.at[q],
                    rows_v.at[g * 16 + l],
                    sem,
                )
            return 0

        lax.fori_loop(0, B_PER_W // 16, body, 0)
        pltpu.make_async_copy(
            out_hbm.at[pl.ds(0, B_PER_W)], rows_v, sem
        ).wait()
        pltpu.sync_copy(rows_v, out_hbm.at[pl.ds(base, B_PER_W)])

    return _sc_gather


_MLP_BLK = 2048


def _mlp_body(hp_ref, xr_ref, w1_ref, b1_ref, w2_ref, b2_ref, o_ref):
    odd = ((xr_ref[...] >> 7) & 1) == 1
    h = jnp.where(odd, hp_ref[:, DIM:], hp_ref[:, :DIM])
    z = jax.lax.dot_general(
        h, w1_ref[...], (((1,), (1,)), ((), ())),
        preferred_element_type=jnp.float32,
    ) + b1_ref[...]
    z = z * jax.nn.sigmoid(z)
    o_ref[...] = jax.lax.dot_general(
        z, w2_ref[...], (((1,), (1,)), ((), ())),
        preferred_element_type=jnp.float32,
    ) + b2_ref[...]


def _mlp(hp, xr, w1, b1, w2, b2):
    grid = (BATCH // _MLP_BLK,)
    return pl.pallas_call(
        _mlp_body,
        grid=grid,
        in_specs=[
            pl.BlockSpec((_MLP_BLK, PAIR), lambda i: (i, 0)),
            pl.BlockSpec((_MLP_BLK, 1), lambda i: (i, 0)),
            pl.BlockSpec((DIM, DIM), lambda i: (0, 0)),
            pl.BlockSpec((1, DIM), lambda i: (0, 0)),
            pl.BlockSpec((DIM, DIM), lambda i: (0, 0)),
            pl.BlockSpec((1, DIM), lambda i: (0, 0)),
        ],
        out_specs=pl.BlockSpec((_MLP_BLK, DIM), lambda i: (i, 0)),
        out_shape=jax.ShapeDtypeStruct((BATCH, DIM), jnp.float32),
    )(hp, xr, w1, b1.reshape(1, DIM), w2, b2.reshape(1, DIM))


def kernel(x, emb, W1, b1, W2, b2):
    idx = x.astype(jnp.int32)
    table = _compact_transpose(emb.T)
    hp = _make_sc_gather()(idx.reshape(NW * N_CHUNK, CHUNK), table)
    return _mlp(hp, idx.reshape(BATCH, 1), W1, b1, W2, b2)


# speedup vs baseline: 2.4929x; 1.0518x over previous
import functools

import jax
import jax.numpy as jnp
from jax import lax
from jax.experimental import pallas as pl
from jax.experimental.pallas import tpu as pltpu
from jax.experimental.pallas import tpu_sc as plsc

DIM = 64
BATCH = 16384
NUM_CLASSES = 1000000
PAIR = 2 * DIM
N_C = 500032

NC = 2
NS = 16
NW = NC * NS
B_PER_W = BATCH // NW
CHUNK = 128
N_CHUNK = B_PER_W // CHUNK

_T_LANES = 32768
_T_GRID = 31


def _transpose_body(xt_ref, o_ref):
    eye = jnp.eye(DIM, dtype=jnp.float32)
    for t in range(_T_LANES // 512):
        y = jax.lax.dot_general(
            xt_ref[:, pl.ds(t * 512, 512)], eye, (((0,), (0,)), ((), ())),
            preferred_element_type=jnp.float32,
        )
        o_ref[pl.ds(t * 256, 128), :] = jnp.concatenate(
            [y[:128], y[128:256]], axis=1
        )
        o_ref[pl.ds(t * 256 + 128, 128), :] = jnp.concatenate(
            [y[256:384], y[384:]], axis=1
        )


def _compact_transpose(emb_t):
    return pl.pallas_call(
        _transpose_body,
        grid=(_T_GRID,),
        in_specs=[pl.BlockSpec((DIM, _T_LANES), lambda m: (0, m))],
        out_specs=pl.BlockSpec((_T_LANES // 2, PAIR), lambda m: (m, 0)),
        compiler_params=pltpu.CompilerParams(
            dimension_semantics=("arbitrary",)
        ),
        out_shape=jax.ShapeDtypeStruct((N_C, PAIR), jnp.float32),
    )(emb_t)


@functools.cache
def _make_sc_gather():
    mesh = plsc.VectorSubcoreMesh(core_axis_name="c", subcore_axis_name="s")

    @functools.partial(
        pl.kernel,
        mesh=mesh,
        out_type=jax.ShapeDtypeStruct((BATCH, PAIR), jnp.float32),
        scratch_types=[
            pltpu.VMEM((N_CHUNK, CHUNK), jnp.int32),
            pltpu.VMEM((B_PER_W, PAIR), jnp.float32),
            pltpu.SemaphoreType.DMA,
        ],
        compiler_params=pltpu.CompilerParams(needs_layout_passes=False),
    )
    def _sc_gather(idx_hbm, table_hbm, out_hbm, idx_v, rows_v, sem):
        wid = lax.axis_index("s") * NC + lax.axis_index("c")
        base = wid * B_PER_W
        pltpu.sync_copy(idx_hbm.at[pl.ds(wid * N_CHUNK, N_CHUNK)], idx_v)
        lane = lax.iota(jnp.int32, 16)

        def body(g, _):
            j = g // 8
            k = (g % 8) * 16
            v = idx_v[j, pl.ds(k, 16)]
            for l in range(16):
                s = jnp.sum(jnp.where(lane == l, v, 0))
                q = ((s >> 8) << 7) | (s & 127)
                pltpu.async_copy(
                    table_hbm.at[q],
                    rows_v.at[g * 16 + l],
                    sem,
                )
            return 0

        lax.fori_loop(0, B_PER_W // 16, body, 0)
        pltpu.make_async_copy(
            out_hbm.at[pl.ds(0, B_PER_W)], rows_v, sem
        ).wait()
        pltpu.sync_copy(rows_v, out_hbm.at[pl.ds(base, B_PER_W)])

    return _sc_gather


_MLP_BLK = 2048


def _mlp_body(hp_ref, xr_ref, w1_ref, b1_ref, w2_ref, b2_ref, o_ref):
    odd = ((xr_ref[...] >> 7) & 1) == 1
    h = jnp.where(odd, hp_ref[:, DIM:], hp_ref[:, :DIM])
    z = jax.lax.dot_general(
        h, w1_ref[...], (((1,), (1,)), ((), ())),
        preferred_element_type=jnp.float32,
    ) + b1_ref[...]
    z = z * jax.nn.sigmoid(z)
    o_ref[...] = jax.lax.dot_general(
        z, w2_ref[...], (((1,), (1,)), ((), ())),
        preferred_element_type=jnp.float32,
    ) + b2_ref[...]


def _mlp(hp, xr, w1, b1, w2, b2):
    grid = (BATCH // _MLP_BLK,)
    return pl.pallas_call(
        _mlp_body,
        grid=grid,
        in_specs=[
            pl.BlockSpec((_MLP_BLK, PAIR), lambda i: (i, 0)),
            pl.BlockSpec((_MLP_BLK, 1), lambda i: (i, 0)),
            pl.BlockSpec((DIM, DIM), lambda i: (0, 0)),
            pl.BlockSpec((1, DIM), lambda i: (0, 0)),
            pl.BlockSpec((DIM, DIM), lambda i: (0, 0)),
            pl.BlockSpec((1, DIM), lambda i: (0, 0)),
        ],
        out_specs=pl.BlockSpec((_MLP_BLK, DIM), lambda i: (i, 0)),
        out_shape=jax.ShapeDtypeStruct((BATCH, DIM), jnp.float32),
    )(hp, xr, w1, b1.reshape(1, DIM), w2, b2.reshape(1, DIM))


def kernel(x, emb, W1, b1, W2, b2):
    idx = x.astype(jnp.int32)
    table = _compact_transpose(emb.T)
    hp = _make_sc_gather()(idx.reshape(NW * N_CHUNK, CHUNK), table)
    return _mlp(hp, idx.reshape(BATCH, 1), W1, b1, W2, b2)
